# baseline (speedup 1.0000x reference)
import functools

import jax
import jax.numpy as jnp
from jax import lax
from jax.experimental import pallas as pl
from jax.experimental.pallas import tpu as pltpu
from jax.experimental.pallas import tpu_sc as plsc

HIDDEN = 128
SCALE = float(HIDDEN) ** 0.5

_NC = 2
_NS = 16
_L = 16
_NW = _NC * _NS


@functools.cache
def _make_gather(B, D, CH):
    b_per_w = B // _NW
    n_ch = b_per_w // CH
    mesh = plsc.VectorSubcoreMesh(core_axis_name="c", subcore_axis_name="s")

    @functools.partial(
        pl.kernel,
        mesh=mesh,
        out_type=jax.ShapeDtypeStruct((B, D), jnp.float32),
        scratch_types=[
            pltpu.VMEM((CH,), jnp.int32),
            pltpu.VMEM((CH, D), jnp.float32),
            pltpu.SemaphoreType.DMA,
        ],
    )
    def k(ids_hbm, table_hbm, out_hbm, idx_v, rows_v, sem):
        wid = lax.axis_index("s") * _NC + lax.axis_index("c")
        base = wid * b_per_w

        def chunk_body(g, carry):
            off = pl.multiple_of(base + g * CH, 8)
            pltpu.sync_copy(ids_hbm.at[pl.ds(off, CH)], idx_v)
            pltpu.async_copy(table_hbm.at[idx_v], rows_v, sem).wait()

            def row_body(r, c2):
                for j in range(D // _L):
                    sl = pl.ds(j * _L, _L)
                    rows_v[r, sl] = rows_v[r, sl] * SCALE
                return c2

            lax.fori_loop(0, CH, row_body, 0)
            pltpu.sync_copy(rows_v, out_hbm.at[pl.ds(off, CH)])
            return carry

        lax.fori_loop(0, n_ch, chunk_body, 0)

    return k


def kernel(input_ids, table):
    ids_flat = input_ids.reshape(-1)
    B = ids_flat.shape[0]
    out = _make_gather(B, HIDDEN, 128)(ids_flat, table)
    return out.reshape(*input_ids.shape, HIDDEN)


# speedup vs baseline: 1.8681x; 1.8681x over previous
import functools

import jax
import jax.numpy as jnp
from jax import lax
from jax.experimental import pallas as pl
from jax.experimental.pallas import tpu as pltpu
from jax.experimental.pallas import tpu_sc as plsc

HIDDEN = 128
SCALE = float(HIDDEN) ** 0.5

_NC = 2
_NS = 16
_L = 16
_NW = _NC * _NS

_CH = 128


@functools.cache
def _make_gather(B, D):
    b_per_w = B // _NW
    n_ch = b_per_w // _CH
    assert b_per_w % _CH == 0 and n_ch % 2 == 0
    mesh = plsc.VectorSubcoreMesh(core_axis_name="c", subcore_axis_name="s")

    @functools.partial(
        pl.kernel,
        mesh=mesh,
        out_type=jax.ShapeDtypeStruct((B, D), jnp.float32),
        scratch_types=[
            pltpu.VMEM((n_ch, _CH), jnp.int32),
            pltpu.VMEM((_CH, D), jnp.float32),
            pltpu.VMEM((_CH, D), jnp.float32),
            pltpu.VMEM((_CH, D), jnp.float32),
            pltpu.VMEM((_CH, D), jnp.float32),
            pltpu.SemaphoreType.DMA,
            pltpu.SemaphoreType.DMA,
            pltpu.SemaphoreType.DMA,
            pltpu.SemaphoreType.DMA,
        ],
    )
    def k(ids_hbm, table_hbm, out_hbm, idx_v, g0, g1, s0, s1,
          semg0, semg1, sems0, sems1):
        gbuf, sbuf = (g0, g1), (s0, s1)
        semg, sems = (semg0, semg1), (sems0, sems1)
        wid = lax.axis_index("s") * _NC + lax.axis_index("c")
        base = wid * b_per_w

        pltpu.sync_copy(ids_hbm.at[wid], idx_v)

        def gather_chunk(c, b):
            return pltpu.make_async_copy(
                table_hbm.at[idx_v.at[c]], gbuf[b], semg[b])

        def scatter_chunk(c, b):
            off = pl.multiple_of(base + c * _CH, 8)
            return pltpu.make_async_copy(
                sbuf[b], out_hbm.at[pl.ds(off, _CH)], sems[b])

        for b in range(2):
            gather_chunk(b, b).start()

        def outer(i, carry):
            for b in range(2):
                c = 2 * i + b
                gather_chunk(c, b).wait()

                @pl.when(i >= 1)
                def _wait_prev_scatter():
                    scatter_chunk(c - 2, b).wait()

                def row_body(r, c2):
                    for j in range(D // _L):
                        sl = pl.ds(j * _L, _L)
                        sbuf[b][r, sl] = gbuf[b][r, sl] * SCALE
                    return c2

                lax.fori_loop(0, _CH, row_body, 0)

                @pl.when(i < n_ch // 2 - 1)
                def _start_next_gather():
                    gather_chunk(c + 2, b).start()

                scatter_chunk(c, b).start()
            return carry

        lax.fori_loop(0, n_ch // 2, outer, 0)

        for b in range(2):
            scatter_chunk(n_ch - 2 + b, b).wait()

    return k


def kernel(input_ids, table):
    ids_flat = input_ids.reshape(-1)
    B = ids_flat.shape[0]
    b_per_w = B // _NW
    ids3 = ids_flat.reshape(_NW, b_per_w // _CH, _CH)
    out = _make_gather(B, HIDDEN)(ids3, table)
    return out.reshape(*input_ids.shape, HIDDEN)
